# baseline (speedup 1.0000x reference)
import jax
import jax.numpy as jnp
from jax.experimental import pallas as pl

B, H, W, C = 8, 24, 24, 384
E = 8
HW = H * W
CC = C * C
NEG = -1e30


def _gate_kernel(x_ref, wg_ref, bg_ref, be_ref, wef_ref,
                 wcomb_ref, bcomb_ref, laux_ref):
    x = x_ref[...].reshape(B, HW, C)
    pooled = jnp.mean(x, axis=1)
    logits = jnp.dot(pooled, wg_ref[...],
                     preferred_element_type=jnp.float32) + bg_ref[...]
    m = jnp.max(logits, axis=1, keepdims=True)
    eg = jnp.exp(logits - m)
    gates = eg / jnp.sum(eg, axis=1, keepdims=True)

    iota = jax.lax.broadcasted_iota(jnp.int32, (B, E), 1)
    m1 = jnp.max(gates, axis=1, keepdims=True)
    i1 = jnp.min(jnp.where(gates == m1, iota, E), axis=1, keepdims=True)
    mask1 = (iota == i1)
    g2 = jnp.where(mask1, NEG, gates)
    m2 = jnp.max(g2, axis=1, keepdims=True)
    i2 = jnp.min(jnp.where(g2 == m2, iota, E), axis=1, keepdims=True)

    e2 = jnp.exp(m2 - m1)
    denom = 1.0 + e2
    w1 = 1.0 / denom
    w2 = e2 / denom
    coeff = jnp.where(mask1, w1, 0.0) + jnp.where(iota == i2, w2, 0.0)

    wcomb_ref[...] = jnp.dot(coeff, wef_ref[...],
                             preferred_element_type=jnp.float32)
    bcomb_ref[...] = jnp.dot(coeff, be_ref[...],
                             preferred_element_type=jnp.float32)

    me = jnp.mean(gates, axis=0, keepdims=True)
    ce = jnp.mean(mask1.astype(jnp.float32), axis=0, keepdims=True)
    laux_ref[...] = jnp.sum(me * ce, axis=1, keepdims=True) * E


def _apply_kernel(x_ref, wc_ref, bc_ref, out_ref):
    x = x_ref[...].reshape(HW, C)
    w = wc_ref[...].reshape(C, C)
    y = jax.lax.dot_general(x, w, (((1,), (1,)), ((), ())),
                            preferred_element_type=jnp.float32)
    y = y + bc_ref[...].reshape(1, C)
    out_ref[...] = y.reshape(1, H, W, C)


def kernel(inputs_raw, W_gate, b_gate, W_experts, b_experts):
    we_flat = W_experts.reshape(E, CC)
    bg = b_gate.reshape(1, E)

    wcomb_flat, bcomb, laux = pl.pallas_call(
        _gate_kernel,
        out_shape=(
            jax.ShapeDtypeStruct((B, CC), jnp.float32),
            jax.ShapeDtypeStruct((B, C), jnp.float32),
            jax.ShapeDtypeStruct((1, 1), jnp.float32),
        ),
    )(inputs_raw, W_gate, bg, b_experts, we_flat)

    wcomb = wcomb_flat.reshape(B, C, C)

    out = pl.pallas_call(
        _apply_kernel,
        grid=(B,),
        in_specs=[
            pl.BlockSpec((1, H, W, C), lambda n: (n, 0, 0, 0)),
            pl.BlockSpec((1, C, C), lambda n: (n, 0, 0)),
            pl.BlockSpec((1, 1, C), lambda n: (n, 0, 0)),
        ],
        out_specs=pl.BlockSpec((1, H, W, C), lambda n: (n, 0, 0, 0)),
        out_shape=jax.ShapeDtypeStruct((B, H, W, C), jnp.float32),
    )(inputs_raw, wcomb, bcomb.reshape(B, 1, C))

    return out, laux[0, 0]


# speedup vs baseline: 2.1685x; 2.1685x over previous
import jax
import jax.numpy as jnp
from jax.experimental import pallas as pl
from jax.experimental.pallas import tpu as pltpu

B, H, W, C = 8, 24, 24, 384
E = 8
HW = H * W
CC = C * C
NEG = -1e30


def _fused_kernel(x_ref, wg_ref, bg_ref, be_ref, we_ref,
                  out_ref, laux_ref, coeff_ref, bcomb_ref, wcomb_ref):
    n = pl.program_id(0)

    @pl.when(n == 0)
    def _gate():
        x = x_ref[...].reshape(B, HW, C)
        pooled = jnp.mean(x, axis=1)
        logits = jnp.dot(pooled, wg_ref[...],
                         preferred_element_type=jnp.float32) + bg_ref[...]
        m = jnp.max(logits, axis=1, keepdims=True)
        eg = jnp.exp(logits - m)
        gates = eg / jnp.sum(eg, axis=1, keepdims=True)

        iota = jax.lax.broadcasted_iota(jnp.int32, (B, E), 1)
        m1 = jnp.max(gates, axis=1, keepdims=True)
        i1 = jnp.min(jnp.where(gates == m1, iota, E), axis=1, keepdims=True)
        mask1 = (iota == i1)
        g2 = jnp.where(mask1, NEG, gates)
        m2 = jnp.max(g2, axis=1, keepdims=True)
        i2 = jnp.min(jnp.where(g2 == m2, iota, E), axis=1, keepdims=True)

        e2 = jnp.exp(m2 - m1)
        denom = 1.0 + e2
        w1 = 1.0 / denom
        w2 = e2 / denom
        coeff = jnp.where(mask1, w1, 0.0) + jnp.where(iota == i2, w2, 0.0)
        coeff_ref[...] = coeff
        bcomb_ref[...] = jnp.dot(coeff, be_ref[...],
                                 preferred_element_type=jnp.float32)

        we_flat = we_ref[...].reshape(E, CC)
        wcomb_ref[...] = jnp.dot(coeff, we_flat,
                                 preferred_element_type=jnp.float32
                                 ).reshape(B, C, C)

        me = jnp.mean(gates, axis=0, keepdims=True)
        ce = jnp.mean(mask1.astype(jnp.float32), axis=0, keepdims=True)
        laux_ref[...] = jnp.sum(me * ce, axis=1, keepdims=True) * E

    x_n = x_ref[pl.ds(n, 1)].reshape(HW, C)
    w_n = wcomb_ref[pl.ds(n, 1)].reshape(C, C)
    y = jax.lax.dot_general(x_n, w_n, (((1,), (1,)), ((), ())),
                            preferred_element_type=jnp.float32)
    y = y + bcomb_ref[pl.ds(n, 1)]
    out_ref[...] = y.reshape(1, H, W, C)


def kernel(inputs_raw, W_gate, b_gate, W_experts, b_experts):
    bg = b_gate.reshape(1, E)

    out, laux = pl.pallas_call(
        _fused_kernel,
        grid=(B,),
        in_specs=[
            pl.BlockSpec((B, H, W, C), lambda n: (0, 0, 0, 0)),
            pl.BlockSpec((C, E), lambda n: (0, 0)),
            pl.BlockSpec((1, E), lambda n: (0, 0)),
            pl.BlockSpec((E, C), lambda n: (0, 0)),
            pl.BlockSpec((E, C, C), lambda n: (0, 0, 0)),
        ],
        out_specs=(
            pl.BlockSpec((1, H, W, C), lambda n: (n, 0, 0, 0)),
            pl.BlockSpec((1, 1), lambda n: (0, 0)),
        ),
        out_shape=(
            jax.ShapeDtypeStruct((B, H, W, C), jnp.float32),
            jax.ShapeDtypeStruct((1, 1), jnp.float32),
        ),
        scratch_shapes=[
            pltpu.VMEM((B, E), jnp.float32),
            pltpu.VMEM((B, C), jnp.float32),
            pltpu.VMEM((B, C, C), jnp.float32),
        ],
    )(inputs_raw, W_gate, bg, b_experts, W_experts)

    return out, laux[0, 0]
